# baseline (speedup 1.0000x reference)
import functools

import jax
import jax.numpy as jnp
from jax import lax
from jax.experimental import pallas as pl
from jax.experimental.pallas import tpu as pltpu
from jax.experimental.pallas import tpu_sc as plsc

N_CLASSES = 1000
N_BATCH = 16384
NUM_CORES = 2
NUM_SUBCORES = 16
LANES = 16
NUM_WORKERS = NUM_CORES * NUM_SUBCORES
ROWS_PER_WORKER = N_BATCH // NUM_WORKERS
CHUNK_ROWS = 64
CHUNK_WORDS = CHUNK_ROWS * N_CLASSES
NUM_CHUNKS = ROWS_PER_WORKER // CHUNK_ROWS
GROUPS = CHUNK_ROWS // LANES


def _onehot_body(x_hbm, ztmpl_hbm, out_hbm, idx_v, buf0, buf1, sem0, sem1):
    wid = lax.axis_index("c") * NUM_SUBCORES + lax.axis_index("s")
    row_base = wid * ROWS_PER_WORKER

    iota16 = lax.iota(jnp.int32, LANES)
    ones = jnp.full((LANES,), 1.0, jnp.float32)
    zeros = jnp.zeros((LANES,), jnp.float32)

    pltpu.sync_copy(x_hbm.at[pl.ds(row_base, ROWS_PER_WORKER)], idx_v)
    pltpu.sync_copy(ztmpl_hbm, buf0)
    pltpu.sync_copy(ztmpl_hbm, buf1)

    bufs = [buf0, buf1]
    sems = [sem0, sem1]
    inflight = [None, None]
    dirty = [None, None]

    for c in range(NUM_CHUNKS):
        b = c % 2
        buf = bufs[b]
        if inflight[b] is not None:
            inflight[b].wait()
            for flat in dirty[b]:
                plsc.store_scatter(buf, [flat], zeros)
        flats = []
        for j in range(GROUPS):
            col = idx_v[pl.ds(c * CHUNK_ROWS + j * LANES, LANES)]
            flat = (iota16 + (j * LANES)) * N_CLASSES + col
            plsc.store_scatter(buf, [flat], ones)
            flats.append(flat)
        dirty[b] = flats
        out_off = (row_base + c * CHUNK_ROWS) * N_CLASSES
        inflight[b] = pltpu.async_copy(
            buf, out_hbm.at[pl.ds(out_off, CHUNK_WORDS)], sems[b])

    inflight[0].wait()
    inflight[1].wait()


_onehot_sc = functools.partial(
    pl.kernel,
    out_type=jax.ShapeDtypeStruct((N_BATCH * N_CLASSES,), jnp.float32),
    mesh=plsc.VectorSubcoreMesh(
        core_axis_name="c", subcore_axis_name="s",
        num_cores=NUM_CORES, num_subcores=NUM_SUBCORES),
    scratch_types=[
        pltpu.VMEM((ROWS_PER_WORKER,), jnp.int32),
        pltpu.VMEM((CHUNK_WORDS,), jnp.float32),
        pltpu.VMEM((CHUNK_WORDS,), jnp.float32),
        pltpu.SemaphoreType.DMA,
        pltpu.SemaphoreType.DMA,
    ],
    compiler_params=pltpu.CompilerParams(needs_layout_passes=False),
)(_onehot_body)


def kernel(x):
    x = x.astype(jnp.int32)
    ztmpl = jnp.zeros((CHUNK_WORDS,), jnp.float32)
    flat = _onehot_sc(x, ztmpl)
    return flat.reshape(N_BATCH, N_CLASSES)


# speedup vs baseline: 1.5426x; 1.5426x over previous
import functools

import jax
import jax.numpy as jnp
from jax import lax
from jax.experimental import pallas as pl
from jax.experimental.pallas import tpu as pltpu
from jax.experimental.pallas import tpu_sc as plsc

N_CLASSES = 1000
N_BATCH = 16384
NUM_CORES = 2
NUM_SUBCORES = 16
LANES = 16
NUM_WORKERS = NUM_CORES * NUM_SUBCORES
ROWS_PER_WORKER = N_BATCH // NUM_WORKERS
CHUNK_ROWS = 32
CHUNK_WORDS = CHUNK_ROWS * N_CLASSES
NUM_CHUNKS = ROWS_PER_WORKER // CHUNK_ROWS
GROUPS = CHUNK_ROWS // LANES


def _onehot_body(x_hbm, ztmpl_hbm, out_hbm, idx_v, buf0, buf1, sem0, sem1):
    wid = lax.axis_index("c") * NUM_SUBCORES + lax.axis_index("s")
    row_base = wid * ROWS_PER_WORKER

    iota16 = lax.iota(jnp.int32, LANES)
    ones = jnp.full((LANES,), 1.0, jnp.float32)
    zeros = jnp.zeros((LANES,), jnp.float32)

    pltpu.sync_copy(x_hbm.at[pl.ds(row_base, ROWS_PER_WORKER)], idx_v)
    pltpu.sync_copy(ztmpl_hbm, buf0)
    pltpu.sync_copy(ztmpl_hbm, buf1)

    bufs = [buf0, buf1]
    sems = [sem0, sem1]
    inflight = [None, None]
    dirty = [None, None]

    for c in range(NUM_CHUNKS):
        b = c % 2
        buf = bufs[b]
        if inflight[b] is not None:
            inflight[b].wait()
            for row, col in dirty[b]:
                plsc.store_scatter(buf, [row, col], zeros)
        flats = []
        for j in range(GROUPS):
            col = idx_v[pl.ds(c * CHUNK_ROWS + j * LANES, LANES)]
            row = iota16 + (j * LANES)
            plsc.store_scatter(buf, [row, col], ones)
            flats.append((row, col))
        dirty[b] = flats
        out_row = row_base + c * CHUNK_ROWS
        inflight[b] = pltpu.async_copy(
            buf, out_hbm.at[pl.ds(out_row, CHUNK_ROWS)], sems[b])

    inflight[0].wait()
    inflight[1].wait()


_onehot_sc = functools.partial(
    pl.kernel,
    out_type=jax.ShapeDtypeStruct((N_BATCH, N_CLASSES), jnp.float32),
    mesh=plsc.VectorSubcoreMesh(
        core_axis_name="c", subcore_axis_name="s",
        num_cores=NUM_CORES, num_subcores=NUM_SUBCORES),
    scratch_types=[
        pltpu.VMEM((ROWS_PER_WORKER,), jnp.int32),
        pltpu.VMEM((CHUNK_ROWS, N_CLASSES), jnp.float32),
        pltpu.VMEM((CHUNK_ROWS, N_CLASSES), jnp.float32),
        pltpu.SemaphoreType.DMA,
        pltpu.SemaphoreType.DMA,
    ],
    compiler_params=pltpu.CompilerParams(needs_layout_passes=False),
)(_onehot_body)


def kernel(x):
    x = x.astype(jnp.int32)
    ztmpl = jnp.zeros((CHUNK_ROWS, N_CLASSES), jnp.float32)
    return _onehot_sc(x, ztmpl)


# speedup vs baseline: 1.5433x; 1.0004x over previous
import functools

import jax
import jax.numpy as jnp
from jax import lax
from jax.experimental import pallas as pl
from jax.experimental.pallas import tpu as pltpu
from jax.experimental.pallas import tpu_sc as plsc

N_CLASSES = 1000
N_BATCH = 16384
NUM_CORES = 2
NUM_SUBCORES = 16
LANES = 16
NUM_WORKERS = NUM_CORES * NUM_SUBCORES
ROWS_PER_WORKER = N_BATCH // NUM_WORKERS
CHUNK_ROWS = 32
CHUNK_WORDS = CHUNK_ROWS * N_CLASSES
NUM_CHUNKS = ROWS_PER_WORKER // CHUNK_ROWS
GROUPS = CHUNK_ROWS // LANES


def _onehot_body(x_hbm, ztmpl_hbm, out_hbm, idx_v, buf0, buf1, sem0, sem1):
    wid = lax.axis_index("c") * NUM_SUBCORES + lax.axis_index("s")
    row_base = wid * ROWS_PER_WORKER

    iota16 = lax.iota(jnp.int32, LANES)
    ones = jnp.full((LANES,), 1.0, jnp.float32)
    zeros = jnp.zeros((LANES,), jnp.float32)

    pltpu.sync_copy(x_hbm.at[pl.ds(row_base, ROWS_PER_WORKER)], idx_v)
    pltpu.sync_copy(ztmpl_hbm, buf0)
    pltpu.sync_copy(ztmpl_hbm, buf1)

    bufs = [buf0, buf1]
    sems = [sem0, sem1]
    inflight = [None, None]
    dirty = [None, None]

    for c in range(NUM_CHUNKS):
        b = c % 2
        buf = bufs[b]
        if inflight[b] is not None:
            inflight[b].wait()
            for row, col in dirty[b]:
                plsc.store_scatter(buf, [row, col], zeros)
        flats = []
        for j in range(GROUPS):
            col = idx_v[pl.ds(c * CHUNK_ROWS + j * LANES, LANES)]
            row = iota16 + (j * LANES)
            plsc.store_scatter(buf, [row, col], ones)
            flats.append((row, col))
        dirty[b] = flats
        out_row = row_base + c * CHUNK_ROWS
        inflight[b] = pltpu.async_copy(
            buf, out_hbm.at[pl.ds(out_row, CHUNK_ROWS)], sems[b])

    inflight[0].wait()
    inflight[1].wait()


_onehot_sc = functools.partial(
    pl.kernel,
    out_type=jax.ShapeDtypeStruct((N_BATCH, N_CLASSES), jnp.float32),
    mesh=plsc.VectorSubcoreMesh(
        core_axis_name="c", subcore_axis_name="s",
        num_cores=NUM_CORES, num_subcores=NUM_SUBCORES),
    scratch_types=[
        pltpu.VMEM((ROWS_PER_WORKER,), jnp.int32),
        pltpu.VMEM((CHUNK_ROWS, N_CLASSES), jnp.float32),
        pltpu.VMEM((CHUNK_ROWS, N_CLASSES), jnp.float32),
        pltpu.SemaphoreType.DMA,
        pltpu.SemaphoreType.DMA,
    ],
    compiler_params=pltpu.CompilerParams(
        needs_layout_passes=False, use_tc_tiling_on_sc=True),
)(_onehot_body)


def kernel(x):
    x = x.astype(jnp.int32)
    ztmpl = jnp.zeros((CHUNK_ROWS, N_CLASSES), jnp.float32)
    return _onehot_sc(x, ztmpl)


# speedup vs baseline: 3.1433x; 2.0368x over previous
import functools

import jax
import jax.numpy as jnp
from jax import lax
from jax.experimental import pallas as pl
from jax.experimental.pallas import tpu as pltpu
from jax.experimental.pallas import tpu_sc as plsc

N_CLASSES = 1000
N_BATCH = 16384
NUM_CORES = 2
NUM_SUBCORES = 16
LANES = 16
NUM_WORKERS = NUM_CORES * NUM_SUBCORES
COLS_PER_WORKER = N_BATCH // NUM_WORKERS
CHUNK_COLS = 128
NUM_CHUNKS = COLS_PER_WORKER // CHUNK_COLS
GROUPS = CHUNK_COLS // LANES


def _onehot_body(x_hbm, ztmpl_hbm, out_hbm, idx_v, buf, sem):
    wid = lax.axis_index("c") * NUM_SUBCORES + lax.axis_index("s")
    col_base = wid * COLS_PER_WORKER

    iota16 = lax.iota(jnp.int32, LANES)
    ones = jnp.full((LANES,), 1.0, jnp.float32)
    zeros = jnp.zeros((LANES,), jnp.float32)

    pltpu.sync_copy(x_hbm.at[pl.ds(col_base, COLS_PER_WORKER)], idx_v)
    pltpu.sync_copy(ztmpl_hbm, buf)

    for c in range(NUM_CHUNKS):
        groups = []
        for j in range(GROUPS):
            cls = idx_v[pl.ds(c * CHUNK_COLS + j * LANES, LANES)]
            col = iota16 + (j * LANES)
            plsc.store_scatter(buf, [cls, col], ones)
            groups.append((cls, col))
        pltpu.async_copy(
            buf, out_hbm.at[:, pl.ds(col_base + c * CHUNK_COLS, CHUNK_COLS)],
            sem).wait()
        if c + 1 < NUM_CHUNKS:
            for cls, col in groups:
                plsc.store_scatter(buf, [cls, col], zeros)


_onehot_sc = functools.partial(
    pl.kernel,
    out_type=jax.ShapeDtypeStruct((N_CLASSES, N_BATCH), jnp.float32),
    mesh=plsc.VectorSubcoreMesh(
        core_axis_name="c", subcore_axis_name="s",
        num_cores=NUM_CORES, num_subcores=NUM_SUBCORES),
    scratch_types=[
        pltpu.VMEM((COLS_PER_WORKER,), jnp.int32),
        pltpu.VMEM((N_CLASSES, CHUNK_COLS), jnp.float32),
        pltpu.SemaphoreType.DMA,
    ],
    compiler_params=pltpu.CompilerParams(
        needs_layout_passes=False, use_tc_tiling_on_sc=True),
)(_onehot_body)


def kernel(x):
    x = x.astype(jnp.int32)
    ztmpl = jnp.zeros((N_CLASSES, CHUNK_COLS), jnp.float32)
    return _onehot_sc(x, ztmpl).T
